# baseline (speedup 1.0000x reference)
import functools

import jax
import jax.numpy as jnp
from jax import lax
from jax.experimental import pallas as pl
from jax.experimental.pallas import tpu as pltpu
from jax.experimental.pallas import tpu_sc as plsc


@functools.lru_cache(maxsize=None)
def _build(bsz, seq, v, d):
    info = plsc.get_sparse_core_info()
    nw = info.num_cores * info.num_subcores
    BB = 128
    assert bsz % (nw * BB) == 0 or bsz == nw * BB
    n_chunks = seq
    assert n_chunks % 2 == 0

    mesh = plsc.VectorSubcoreMesh(core_axis_name="c", subcore_axis_name="s")

    @functools.partial(
        pl.kernel,
        mesh=mesh,
        out_type=jax.ShapeDtypeStruct((seq, d, bsz), jnp.float32),
        scratch_types=[
            pltpu.VMEM((BB,), jnp.int32),
            pltpu.VMEM((BB,), jnp.int32),
            pltpu.VMEM((BB, 128), jnp.float32),
            pltpu.VMEM((BB, 128), jnp.float32),
            pltpu.VMEM((d, BB), jnp.float32),
            pltpu.VMEM((d, BB), jnp.float32),
            pltpu.SemaphoreType.DMA,
            pltpu.SemaphoreType.DMA,
            pltpu.SemaphoreType.DMA,
            pltpu.SemaphoreType.DMA,
            pltpu.SemaphoreType.DMA,
            pltpu.SemaphoreType.DMA,
        ],
        compiler_params=pltpu.CompilerParams(use_tc_tiling_on_sc=True, needs_layout_passes=False),
    )
    def k(table_hbm, ids_hbm, out_hbm,
          idx0, idx1, gin0, gin1, gout0, gout1,
          sem_i0, sem_i1, sem_g0, sem_g1, sem_s0, sem_s1):
        idx = (idx0, idx1)
        gin = (gin0, gin1)
        gout = (gout0, gout1)
        sem_i = (sem_i0, sem_i1)
        sem_g = (sem_g0, sem_g1)
        sem_s = (sem_s0, sem_s1)
        wid = lax.axis_index("s") * info.num_cores + lax.axis_index("c")
        b0 = wid * BB
        it16 = lax.iota(jnp.int32, 16)

        def idx_copy(i, b):
            return pltpu.make_async_copy(
                ids_hbm.at[i, pl.ds(b0, BB)], idx[b], sem_i[b])

        def gather_copy(b):
            return pltpu.make_async_copy(
                table_hbm.at[idx[b]], gin[b], sem_g[b])

        def store_copy(i, b):
            return pltpu.make_async_copy(
                gout[b], out_hbm.at[i, :, pl.ds(b0, BB)], sem_s[b])

        def transpose(b):
            gi, go = gin[b], gout[b]

            def body(e, carry):
                for c in range(8):
                    rows = it16 + (c * 16)
                    cols = it16 * 0 + e
                    val = plsc.load_gather(gi, [rows, cols])
                    go.at[e][pl.ds(c * 16, 16)] = val
                return carry

            lax.fori_loop(0, d, body, 0)

        idx_copy(0, 0).start()
        idx_copy(1, 1).start()
        idx_copy(0, 0).wait()
        gather_copy(0).start()

        def body(g, carry):
            for b in (0, 1):
                i = 2 * g + b
                ob = 1 - b

                @pl.when(i + 1 < n_chunks)
                def _():
                    idx_copy(i + 1, ob).wait()

                gather_copy(b).wait()

                @pl.when(i + 1 < n_chunks)
                def _():
                    gather_copy(ob).start()

                @pl.when(i + 2 < n_chunks)
                def _():
                    idx_copy(i + 2, b).start()

                @pl.when(i >= 2)
                def _():
                    store_copy(i - 2, b).wait()

                transpose(b)
                store_copy(i, b).start()
            return carry

        lax.fori_loop(0, n_chunks // 2, body, 0)
        store_copy(n_chunks - 2, 0).wait()
        store_copy(n_chunks - 1, 1).wait()

    return k


def kernel(input_ids, table):
    bsz, seq = input_ids.shape
    v, d = table.shape
    ids_t = input_ids.T.astype(jnp.int32)
    table_pad = jnp.pad(table, ((0, 0), (0, 128 - d)))
    out_t = _build(bsz, seq, v, d)(table_pad, ids_t)
    return (out_t.transpose(2, 0, 1),)


# speedup vs baseline: 1.1316x; 1.1316x over previous
import functools

import jax
import jax.numpy as jnp
from jax import lax
from jax.experimental import pallas as pl
from jax.experimental.pallas import tpu as pltpu
from jax.experimental.pallas import tpu_sc as plsc


@functools.lru_cache(maxsize=None)
def _build(bsz, seq, v, d):
    info = plsc.get_sparse_core_info()
    nw = info.num_cores * info.num_subcores
    BB = 128
    assert bsz % (nw * BB) == 0 or bsz == nw * BB
    n_chunks = seq
    assert n_chunks % 2 == 0

    mesh = plsc.VectorSubcoreMesh(core_axis_name="c", subcore_axis_name="s")

    @functools.partial(
        pl.kernel,
        mesh=mesh,
        out_type=jax.ShapeDtypeStruct((seq, d, bsz), jnp.float32),
        scratch_types=[
            pltpu.VMEM((BB,), jnp.int32),
            pltpu.VMEM((BB,), jnp.int32),
            pltpu.VMEM((BB, 128), jnp.float32),
            pltpu.VMEM((BB, 128), jnp.float32),
            pltpu.VMEM((d, BB), jnp.float32),
            pltpu.VMEM((d, BB), jnp.float32),
            pltpu.SemaphoreType.DMA,
            pltpu.SemaphoreType.DMA,
            pltpu.SemaphoreType.DMA,
            pltpu.SemaphoreType.DMA,
            pltpu.SemaphoreType.DMA,
            pltpu.SemaphoreType.DMA,
        ],
        compiler_params=pltpu.CompilerParams(use_tc_tiling_on_sc=True, needs_layout_passes=False),
    )
    def k(table_hbm, ids_hbm, out_hbm,
          idx0, idx1, gin0, gin1, gout0, gout1,
          sem_i0, sem_i1, sem_g0, sem_g1, sem_s0, sem_s1):
        idx = (idx0, idx1)
        gin = (gin0, gin1)
        gout = (gout0, gout1)
        sem_i = (sem_i0, sem_i1)
        sem_g = (sem_g0, sem_g1)
        sem_s = (sem_s0, sem_s1)
        wid = lax.axis_index("s") * info.num_cores + lax.axis_index("c")
        b0 = wid * BB
        it16 = lax.iota(jnp.int32, 16)

        def idx_copy(i, b):
            return pltpu.make_async_copy(
                ids_hbm.at[i, pl.ds(b0, BB)], idx[b], sem_i[b])

        def gather_copy(b):
            return pltpu.make_async_copy(
                table_hbm.at[idx[b]], gin[b], sem_g[b])

        def store_copy(i, b):
            return pltpu.make_async_copy(
                gout[b], out_hbm.at[i, :, pl.ds(b0, BB)], sem_s[b])

        rowvecs = [it16 + (c * 16) for c in range(d // 16)]

        def transpose(b):
            gi, go = gin[b], gout[b]
            for r in range(BB):
                colvec = it16 * 0 + r
                for c in range(d // 16):
                    val = gi[r, pl.ds(c * 16, 16)]
                    plsc.store_scatter(go, [rowvecs[c], colvec], val)

        idx_copy(0, 0).start()
        idx_copy(1, 1).start()
        idx_copy(0, 0).wait()
        gather_copy(0).start()

        def body(g, carry):
            for b in (0, 1):
                i = 2 * g + b
                ob = 1 - b

                @pl.when(i + 1 < n_chunks)
                def _():
                    idx_copy(i + 1, ob).wait()

                gather_copy(b).wait()

                @pl.when(i + 1 < n_chunks)
                def _():
                    gather_copy(ob).start()

                @pl.when(i + 2 < n_chunks)
                def _():
                    idx_copy(i + 2, b).start()

                @pl.when(i >= 2)
                def _():
                    store_copy(i - 2, b).wait()

                transpose(b)
                store_copy(i, b).start()
            return carry

        lax.fori_loop(0, n_chunks // 2, body, 0)
        store_copy(n_chunks - 2, 0).wait()
        store_copy(n_chunks - 1, 1).wait()

    return k


def kernel(input_ids, table):
    bsz, seq = input_ids.shape
    v, d = table.shape
    ids_t = input_ids.T.astype(jnp.int32)
    table_pad = jnp.pad(table, ((0, 0), (0, 128 - d)))
    out_t = _build(bsz, seq, v, d)(table_pad, ids_t)
    return (out_t.transpose(2, 0, 1),)


# speedup vs baseline: 1.1345x; 1.0026x over previous
import functools

import jax
import jax.numpy as jnp
from jax import lax
from jax.experimental import pallas as pl
from jax.experimental.pallas import tpu as pltpu
from jax.experimental.pallas import tpu_sc as plsc


@functools.lru_cache(maxsize=None)
def _build(bsz, seq, v, d):
    info = plsc.get_sparse_core_info()
    nw = info.num_cores * info.num_subcores
    BB = 128
    assert bsz % (nw * BB) == 0 or bsz == nw * BB
    n_chunks = seq
    assert n_chunks % 2 == 0

    mesh = plsc.VectorSubcoreMesh(core_axis_name="c", subcore_axis_name="s")

    @functools.partial(
        pl.kernel,
        mesh=mesh,
        out_type=jax.ShapeDtypeStruct((seq, d, bsz), jnp.float32),
        scratch_types=[
            pltpu.VMEM((BB,), jnp.int32),
            pltpu.VMEM((BB,), jnp.int32),
            pltpu.VMEM((BB, 128), jnp.float32),
            pltpu.VMEM((BB, 128), jnp.float32),
            pltpu.VMEM((d, BB), jnp.float32),
            pltpu.VMEM((d, BB), jnp.float32),
            pltpu.SemaphoreType.DMA,
            pltpu.SemaphoreType.DMA,
            pltpu.SemaphoreType.DMA,
            pltpu.SemaphoreType.DMA,
            pltpu.SemaphoreType.DMA,
            pltpu.SemaphoreType.DMA,
        ],
        compiler_params=pltpu.CompilerParams(use_tc_tiling_on_sc=True, needs_layout_passes=False),
    )
    def k(table_hbm, ids_hbm, out_hbm,
          idx0, idx1, gin0, gin1, gout0, gout1,
          sem_i0, sem_i1, sem_g0, sem_g1, sem_s0, sem_s1):
        idx = (idx0, idx1)
        gin = (gin0, gin1)
        gout = (gout0, gout1)
        sem_i = (sem_i0, sem_i1)
        sem_g = (sem_g0, sem_g1)
        sem_s = (sem_s0, sem_s1)
        wid = lax.axis_index("s") * info.num_cores + lax.axis_index("c")
        b0 = wid * BB
        it16 = lax.iota(jnp.int32, 16)

        def idx_copy(i, b):
            return pltpu.make_async_copy(
                ids_hbm.at[i, pl.ds(b0, BB)], idx[b], sem_i[b])

        def gather_copy(b):
            return pltpu.make_async_copy(
                table_hbm.at[idx[b]], gin[b], sem_g[b])

        def store_copy(i, b):
            return pltpu.make_async_copy(
                gout[b], out_hbm.at[i, :, pl.ds(b0, BB)], sem_s[b])

        rowvecs = [it16 + (c * 16) for c in range(d // 16)]

        def transpose(b):
            gi, go = gin[b], gout[b]
            nc = d // 16
            for r0 in range(0, BB, 4):
                grp = []
                for rr in range(4):
                    colvec = it16 * 0 + (r0 + rr)
                    for c in range(nc):
                        grp.append((gi[r0 + rr, pl.ds(c * 16, 16)],
                                    rowvecs[c], colvec))
                for val, rv, cv in grp:
                    plsc.store_scatter(go, [rv, cv], val)

        idx_copy(0, 0).start()
        idx_copy(1, 1).start()
        idx_copy(0, 0).wait()
        gather_copy(0).start()

        def body(g, carry):
            for b in (0, 1):
                i = 2 * g + b
                ob = 1 - b

                @pl.when(i + 1 < n_chunks)
                def _():
                    idx_copy(i + 1, ob).wait()

                gather_copy(b).wait()

                @pl.when(i + 1 < n_chunks)
                def _():
                    gather_copy(ob).start()

                @pl.when(i + 2 < n_chunks)
                def _():
                    idx_copy(i + 2, b).start()

                @pl.when(i >= 2)
                def _():
                    store_copy(i - 2, b).wait()

                transpose(b)
                store_copy(i, b).start()
            return carry

        lax.fori_loop(0, n_chunks // 2, body, 0)
        store_copy(n_chunks - 2, 0).wait()
        store_copy(n_chunks - 1, 1).wait()

    return k


def kernel(input_ids, table):
    bsz, seq = input_ids.shape
    v, d = table.shape
    ids_t = input_ids.T.astype(jnp.int32)
    table_pad = jnp.pad(table, ((0, 0), (0, 128 - d)))
    out_t = _build(bsz, seq, v, d)(table_pad, ids_t)
    return (out_t.transpose(2, 0, 1),)


# speedup vs baseline: 1.1356x; 1.0009x over previous
import functools

import jax
import jax.numpy as jnp
from jax import lax
from jax.experimental import pallas as pl
from jax.experimental.pallas import tpu as pltpu
from jax.experimental.pallas import tpu_sc as plsc

NB = 4


@functools.lru_cache(maxsize=None)
def _build(bsz, seq, v, d):
    info = plsc.get_sparse_core_info()
    nw = info.num_cores * info.num_subcores
    BB = bsz // nw
    assert BB == 128
    n_chunks = seq
    assert n_chunks % NB == 0

    mesh = plsc.VectorSubcoreMesh(core_axis_name="c", subcore_axis_name="s")

    @functools.partial(
        pl.kernel,
        mesh=mesh,
        out_type=jax.ShapeDtypeStruct((seq, d, bsz), jnp.float32),
        scratch_types=[
            pltpu.VMEM((NB, BB), jnp.int32),
            pltpu.VMEM((NB, BB, 128), jnp.float32),
            pltpu.VMEM((NB, d, BB), jnp.float32),
            [pltpu.SemaphoreType.DMA] * NB,
            [pltpu.SemaphoreType.DMA] * NB,
            [pltpu.SemaphoreType.DMA] * NB,
        ],
        compiler_params=pltpu.CompilerParams(
            use_tc_tiling_on_sc=True, needs_layout_passes=False),
    )
    def k(table_hbm, ids_hbm, out_hbm, idx, gin, gout, sem_i, sem_g, sem_s):
        wid = lax.axis_index("s") * info.num_cores + lax.axis_index("c")
        b0 = wid * BB
        it16 = lax.iota(jnp.int32, 16)
        rowvecs = [it16 + (c * 16) for c in range(d // 16)]

        def idx_copy(i, b):
            return pltpu.make_async_copy(
                ids_hbm.at[i, pl.ds(b0, BB)], idx.at[b], sem_i[b])

        def gather_copy(b):
            return pltpu.make_async_copy(
                table_hbm.at[idx.at[b]], gin.at[b], sem_g[b])

        def store_copy(i, b):
            return pltpu.make_async_copy(
                gout.at[b], out_hbm.at[i, :, pl.ds(b0, BB)], sem_s[b])

        def transpose(b):
            nc = d // 16
            for r0 in range(0, BB, 4):
                grp = []
                for rr in range(4):
                    colvec = it16 * 0 + (r0 + rr)
                    for c in range(nc):
                        grp.append((gin[b, r0 + rr, pl.ds(c * 16, 16)],
                                    rowvecs[c], colvec))
                for val, rv, cv in grp:
                    plsc.store_scatter(gout.at[b], [rv, cv], val)

        for j in range(NB):
            idx_copy(j, j).start()
        idx_copy(0, 0).wait()
        gather_copy(0).start()
        idx_copy(1, 1).wait()
        gather_copy(1).start()

        def body(g, carry):
            for b in range(NB):
                i = NB * g + b

                gather_copy(b).wait()

                @pl.when(i + 2 < n_chunks)
                def _():
                    b2 = (b + 2) % NB
                    idx_copy(i + 2, b2).wait()
                    gather_copy(b2).start()

                @pl.when(i + NB < n_chunks)
                def _():
                    idx_copy(i + NB, b).start()

                @pl.when(i >= NB)
                def _():
                    store_copy(i - NB, b).wait()

                transpose(b)
                store_copy(i, b).start()
            return carry

        lax.fori_loop(0, n_chunks // NB, body, 0)
        for j in range(NB):
            store_copy(n_chunks - NB + j, j).wait()

    return k


def kernel(input_ids, table):
    bsz, seq = input_ids.shape
    v, d = table.shape
    ids_t = input_ids.T.astype(jnp.int32)
    table_pad = jnp.pad(table, ((0, 0), (0, 128 - d)))
    out_t = _build(bsz, seq, v, d)(table_pad, ids_t)
    return (out_t.transpose(2, 0, 1),)
